# baseline (speedup 1.0000x reference)
import functools

import jax
import jax.numpy as jnp
import numpy as np
from jax import lax
from jax.experimental import pallas as pl
from jax.experimental.pallas import tpu as pltpu
from jax.experimental.pallas import tpu_sc as plsc

B, S, D = 2, 2048, 1024
H = 16
DH = D // H
E, K, DFF = 8, 2, 2048
CAP = int(np.ceil(K * B * S / E * 1.25))
EPS = 1e-6
T = B * S
NCH = K * T
RB = 512
NBLK = T // RB
BQ = 512
BC = 256
NSLOT = E * CAP + BC
NFB = NSLOT // BC
NW = 32
CPW = NCH // NW
CHUNK = 64
NCHUNK = CPW // CHUNK


def _dot(a, b):
    return lax.dot_general(a, b, (((a.ndim - 1,), (0,)), ((), ())),
                           preferred_element_type=jnp.float32)


def _qkv_body(x_ref, w_ref, g_ref, xn_ref, qkv_ref):
    x = x_ref[...]
    nrm = lax.rsqrt(jnp.mean(x * x, axis=1, keepdims=True) + EPS)
    xn = x * nrm * g_ref[...]
    xn_ref[...] = xn
    qkv_ref[...] = _dot(xn, w_ref[...])


def _qkv(x2d, wqkv, rms1):
    return pl.pallas_call(
        _qkv_body,
        grid=(NBLK,),
        in_specs=[
            pl.BlockSpec((RB, D), lambda i: (i, 0)),
            pl.BlockSpec((D, 3 * D), lambda i: (0, 0)),
            pl.BlockSpec((1, D), lambda i: (0, 0)),
        ],
        out_specs=[
            pl.BlockSpec((RB, D), lambda i: (i, 0)),
            pl.BlockSpec((RB, 3 * D), lambda i: (i, 0)),
        ],
        out_shape=[
            jax.ShapeDtypeStruct((T, D), jnp.float32),
            jax.ShapeDtypeStruct((T, 3 * D), jnp.float32),
        ],
    )(x2d, wqkv, rms1)


def _attn_body(q_ref, k_ref, v_ref, o_ref):
    qi = pl.program_id(1)
    q = q_ref[0, 0]
    k = k_ref[0, 0]
    v = v_ref[0, 0]
    s = lax.dot_general(q, k, (((1,), (1,)), ((), ())),
                        preferred_element_type=jnp.float32)
    s = s * (1.0 / np.sqrt(DH))
    row = qi * BQ + lax.broadcasted_iota(jnp.int32, (BQ, S), 0)
    col = lax.broadcasted_iota(jnp.int32, (BQ, S), 1)
    s = jnp.where(col <= row, s, jnp.float32(-1e9))
    m = jnp.max(s, axis=1, keepdims=True)
    p = jnp.exp(s - m)
    l = jnp.sum(p, axis=1, keepdims=True)
    o = lax.dot_general(p, v, (((1,), (0,)), ((), ())),
                        preferred_element_type=jnp.float32)
    o_ref[0, 0] = o / l


def _attention(q4, k4, v4):
    return pl.pallas_call(
        _attn_body,
        grid=(B * H, S // BQ),
        in_specs=[
            pl.BlockSpec((1, 1, BQ, DH), lambda bh, qi: (bh // H, bh % H, qi, 0)),
            pl.BlockSpec((1, 1, S, DH), lambda bh, qi: (bh // H, bh % H, 0, 0)),
            pl.BlockSpec((1, 1, S, DH), lambda bh, qi: (bh // H, bh % H, 0, 0)),
        ],
        out_specs=pl.BlockSpec((1, 1, BQ, DH), lambda bh, qi: (bh // H, bh % H, qi, 0)),
        out_shape=jax.ShapeDtypeStruct((B, H, S, DH), jnp.float32),
    )(q4, k4, v4)


def _oproj_body(o_ref, xn_ref, w_ref, g_ref, hid_ref):
    acc = _dot(o_ref[0, 0], w_ref[0])
    for h in range(1, H):
        acc = acc + _dot(o_ref[0, h], w_ref[h])
    t = acc + xn_ref[...]
    nrm = lax.rsqrt(jnp.mean(t * t, axis=1, keepdims=True) + EPS)
    hid_ref[...] = t * nrm * g_ref[...]


def _oproj(o4, xn, wo_r, rms2):
    nb_per_b = S // RB
    return pl.pallas_call(
        _oproj_body,
        grid=(NBLK,),
        in_specs=[
            pl.BlockSpec((1, H, RB, DH), lambda i: (i // nb_per_b, 0, i % nb_per_b, 0)),
            pl.BlockSpec((RB, D), lambda i: (i, 0)),
            pl.BlockSpec((H, DH, D), lambda i: (0, 0, 0)),
            pl.BlockSpec((1, D), lambda i: (0, 0)),
        ],
        out_specs=pl.BlockSpec((RB, D), lambda i: (i, 0)),
        out_shape=jax.ShapeDtypeStruct((T, D), jnp.float32),
    )(o4, xn, wo_r, rms2)


def _router_body(h_ref, wr_ref, st_ref, dest_ref, gate_ref, stats_ref, carry_ref):
    k = pl.program_id(0)
    rb = pl.program_id(1)

    @pl.when(jnp.logical_and(k == 0, rb == 0))
    def _init():
        carry_ref[...] = jnp.zeros_like(carry_ref)

    h = h_ref[...]
    logits = _dot(h, wr_ref[...])
    lane = lax.broadcasted_iota(jnp.int32, (RB, 128), 1)
    valid = lane < E
    lg = jnp.where(valid, logits, jnp.float32(-1e30))
    m = jnp.max(lg, axis=1, keepdims=True)
    p = jnp.exp(lg - m) * valid.astype(jnp.float32)
    probs = p / jnp.sum(p, axis=1, keepdims=True)

    m0 = jnp.max(probs, axis=1, keepdims=True)
    oh0 = jnp.logical_and(probs == m0, valid)
    e0 = jnp.min(jnp.where(oh0, lane, 128), axis=1, keepdims=True)
    probs2 = jnp.where(lane == e0, jnp.float32(-1.0), probs)
    m1 = jnp.max(probs2, axis=1, keepdims=True)
    oh1 = probs2 == m1
    e1 = jnp.min(jnp.where(oh1, lane, 128), axis=1, keepdims=True)

    is0 = k == 0
    e_sel = jnp.where(is0, e0, e1)
    g = jnp.where(is0, m0, m1) / (m0 + m1)
    oh = (lane == e_sel).astype(jnp.float32)

    ii = lax.broadcasted_iota(jnp.int32, (RB, RB), 0)
    jj = lax.broadcasted_iota(jnp.int32, (RB, RB), 1)
    tril = (jj <= ii).astype(jnp.float32)
    rcnt = lax.dot_general(tril, oh, (((1,), (0,)), ((), ())),
                           precision=lax.Precision.HIGHEST,
                           preferred_element_type=jnp.float32)
    carry = carry_ref[...]
    pos = jnp.sum((rcnt + carry) * oh, axis=1, keepdims=True) - 1.0
    carry_ref[...] = carry + jnp.sum(oh, axis=0, keepdims=True)

    keep = pos < CAP
    pos_i = jnp.minimum(pos, CAP - 1).astype(jnp.int32)
    dest = jnp.where(keep, e_sel * CAP + pos_i, E * CAP)
    dest_ref[0] = jnp.broadcast_to(dest, (RB, 128))
    gate_ref[0] = jnp.broadcast_to(g, (RB, 128))

    @pl.when(jnp.logical_and(k == 1, rb == NBLK - 1))
    def _fin():
        stats_ref[...] = st_ref[...] + jnp.minimum(carry_ref[...], float(CAP))


def _router(hidden, wr_pad, stats_pad):
    return pl.pallas_call(
        _router_body,
        grid=(K, NBLK),
        in_specs=[
            pl.BlockSpec((RB, D), lambda k, rb: (rb, 0)),
            pl.BlockSpec((D, 128), lambda k, rb: (0, 0)),
            pl.BlockSpec((1, 128), lambda k, rb: (0, 0)),
        ],
        out_specs=[
            pl.BlockSpec((1, RB, 128), lambda k, rb: (k, rb, 0)),
            pl.BlockSpec((1, RB, 128), lambda k, rb: (k, rb, 0)),
            pl.BlockSpec((1, 128), lambda k, rb: (0, 0)),
        ],
        out_shape=[
            jax.ShapeDtypeStruct((K, T, 128), jnp.int32),
            jax.ShapeDtypeStruct((K, T, 128), jnp.float32),
            jax.ShapeDtypeStruct((1, 128), jnp.float32),
        ],
        scratch_shapes=[pltpu.VMEM((1, 128), jnp.float32)],
    )(hidden, wr_pad, stats_pad)


_SC_SCRATCH = [
    pltpu.VMEM((NCHUNK, CHUNK), jnp.int32),
    pltpu.VMEM((CHUNK, D), jnp.float32),
    pltpu.SemaphoreType.DMA,
]


def _sc_mesh():
    return plsc.VectorSubcoreMesh(core_axis_name="c", subcore_axis_name="s")


def _sc_dispatch(hidden, dest3):
    def body(hid, dst, buf, idx_v, rows_v, sem):
        wid = lax.axis_index("s") * 2 + lax.axis_index("c")
        pltpu.sync_copy(dst.at[wid], idx_v)
        tok = lax.rem(wid, 16) * CPW
        for c in range(NCHUNK):
            pltpu.sync_copy(hid.at[pl.ds(tok + c * CHUNK, CHUNK)], rows_v)
            pltpu.async_copy(rows_v, buf.at[idx_v.at[c]], sem).wait()

    return pl.kernel(
        body,
        out_type=jax.ShapeDtypeStruct((NSLOT, D), jnp.float32),
        mesh=_sc_mesh(),
        scratch_types=_SC_SCRATCH,
    )(hidden, dest3)


def _sc_combine(outbuf, dest3):
    def body(obuf, dst, gath, idx_v, rows_v, sem):
        wid = lax.axis_index("s") * 2 + lax.axis_index("c")
        pltpu.sync_copy(dst.at[wid], idx_v)
        for c in range(NCHUNK):
            pltpu.async_copy(obuf.at[idx_v.at[c]], rows_v, sem).wait()
            pltpu.sync_copy(rows_v, gath.at[pl.ds(wid * CPW + c * CHUNK, CHUNK)])

    return pl.kernel(
        body,
        out_type=jax.ShapeDtypeStruct((NCH, D), jnp.float32),
        mesh=_sc_mesh(),
        scratch_types=_SC_SCRATCH,
    )(outbuf, dest3)


def _ffn_body(x_ref, w1_ref, w2_ref, o_ref):
    h = jnp.maximum(_dot(x_ref[...], w1_ref[0]), 0.0)
    y = _dot(h, w2_ref[0])
    o_ref[...] = jnp.where(pl.program_id(0) == NFB - 1, 0.0, y)


def _ffn(buf, w1, w2):
    return pl.pallas_call(
        _ffn_body,
        grid=(NFB,),
        in_specs=[
            pl.BlockSpec((BC, D), lambda i: (i, 0)),
            pl.BlockSpec((1, D, DFF), lambda i: (jnp.minimum(i * BC // CAP, E - 1), 0, 0)),
            pl.BlockSpec((1, DFF, D), lambda i: (jnp.minimum(i * BC // CAP, E - 1), 0, 0)),
        ],
        out_specs=pl.BlockSpec((BC, D), lambda i: (i, 0)),
        out_shape=jax.ShapeDtypeStruct((NSLOT, D), jnp.float32),
    )(buf, w1, w2)


def _combine_body(h_ref, r0_ref, r1_ref, g0_ref, g1_ref, o_ref):
    g0 = g0_ref[0][:, 0:1]
    g1 = g1_ref[0][:, 0:1]
    o_ref[...] = (2.0 * h_ref[...] + g0 * r0_ref[...] + g1 * r1_ref[...])


def _final(hidden, gathered, gates_b):
    return pl.pallas_call(
        _combine_body,
        grid=(NBLK,),
        in_specs=[
            pl.BlockSpec((RB, D), lambda i: (i, 0)),
            pl.BlockSpec((RB, D), lambda i: (i, 0)),
            pl.BlockSpec((RB, D), lambda i: (i + NBLK, 0)),
            pl.BlockSpec((1, RB, 128), lambda i: (0, i, 0)),
            pl.BlockSpec((1, RB, 128), lambda i: (1, i, 0)),
        ],
        out_specs=pl.BlockSpec((RB, D), lambda i: (i, 0)),
        out_shape=jax.ShapeDtypeStruct((T, D), jnp.float32),
    )(hidden, gathered, gathered, gates_b, gates_b)


def kernel(tokens, stats, rms1_w, rms2_w, Wq, Wk, Wv, Wo, Wr, W1, W2):
    x2d = tokens.reshape(T, D)
    wqkv = jnp.concatenate([Wq, Wk, Wv], axis=1)
    xn, qkv = _qkv(x2d, wqkv, rms1_w.reshape(1, D))

    q4 = qkv[:, :D].reshape(B, S, H, DH).transpose(0, 2, 1, 3)
    k4 = qkv[:, D:2 * D].reshape(B, S, H, DH).transpose(0, 2, 1, 3)
    v4 = qkv[:, 2 * D:].reshape(B, S, H, DH).transpose(0, 2, 1, 3)
    o4 = _attention(q4, k4, v4)

    wo_r = Wo.reshape(H, DH, D)
    hidden = _oproj(o4, xn, wo_r, rms2_w.reshape(1, D))

    wr_pad = jnp.pad(Wr, ((0, 0), (0, 128 - E)))
    stats_pad = jnp.pad(stats, (0, 128 - E)).reshape(1, 128)
    dest_b, gates_b, stats_out = _router(hidden, wr_pad, stats_pad)

    dest3 = dest_b[:, :, 0].reshape(NW, NCHUNK, CHUNK)
    buf = _sc_dispatch(hidden, dest3)
    outbuf = _ffn(buf, W1, W2)
    gathered = _sc_combine(outbuf, dest3)

    out2d = _final(hidden, gathered, gates_b)
    return out2d.reshape(B, S, D), stats_out[0, :E]


# speedup vs baseline: 1.0125x; 1.0125x over previous
import functools

import jax
import jax.numpy as jnp
import numpy as np
from jax import lax
from jax.experimental import pallas as pl
from jax.experimental.pallas import tpu as pltpu
from jax.experimental.pallas import tpu_sc as plsc

B, S, D = 2, 2048, 1024
H = 16
DH = D // H
E, K, DFF = 8, 2, 2048
CAP = int(np.ceil(K * B * S / E * 1.25))
EPS = 1e-6
T = B * S
NCH = K * T
RB = 512
NBLK = T // RB
BQ = 512
BC = 256
NSLOT = E * CAP + BC
NFB = NSLOT // BC
NW = 32
CPW = NCH // NW
CHUNK = 64
NCHUNK = CPW // CHUNK


def _dot(a, b):
    return lax.dot_general(a, b, (((a.ndim - 1,), (0,)), ((), ())),
                           preferred_element_type=jnp.float32)


def _qkv_body(x_ref, w_ref, g_ref, xn_ref, qkv_ref):
    x = x_ref[...]
    nrm = lax.rsqrt(jnp.mean(x * x, axis=1, keepdims=True) + EPS)
    xn = x * nrm * g_ref[...]
    xn_ref[...] = xn
    qkv_ref[...] = _dot(xn.astype(jnp.bfloat16), w_ref[...]).astype(jnp.bfloat16)


def _qkv(x2d, wqkv, rms1):
    return pl.pallas_call(
        _qkv_body,
        grid=(NBLK,),
        in_specs=[
            pl.BlockSpec((RB, D), lambda i: (i, 0)),
            pl.BlockSpec((D, 3 * D), lambda i: (0, 0)),
            pl.BlockSpec((1, D), lambda i: (0, 0)),
        ],
        out_specs=[
            pl.BlockSpec((RB, D), lambda i: (i, 0)),
            pl.BlockSpec((RB, 3 * D), lambda i: (i, 0)),
        ],
        out_shape=[
            jax.ShapeDtypeStruct((T, D), jnp.float32),
            jax.ShapeDtypeStruct((T, 3 * D), jnp.bfloat16),
        ],
    )(x2d, wqkv, rms1)


def _attn_body(q_ref, k_ref, v_ref, o_ref):
    qi = pl.program_id(1)
    q = q_ref[0, 0]
    k = k_ref[0, 0]
    v = v_ref[0, 0]
    s = lax.dot_general(q, k, (((1,), (1,)), ((), ())),
                        preferred_element_type=jnp.float32)
    s = s * (1.0 / np.sqrt(DH))
    row = qi * BQ + lax.broadcasted_iota(jnp.int32, (BQ, S), 0)
    col = lax.broadcasted_iota(jnp.int32, (BQ, S), 1)
    s = jnp.where(col <= row, s, jnp.float32(-1e9))
    m = jnp.max(s, axis=1, keepdims=True)
    p = jnp.exp(s - m)
    l = jnp.sum(p, axis=1, keepdims=True)
    o = lax.dot_general(p.astype(jnp.bfloat16), v, (((1,), (0,)), ((), ())),
                        preferred_element_type=jnp.float32)
    o_ref[0, 0] = (o / l).astype(jnp.bfloat16)


def _attention(q4, k4, v4):
    return pl.pallas_call(
        _attn_body,
        grid=(B * H, S // BQ),
        in_specs=[
            pl.BlockSpec((1, 1, BQ, DH), lambda bh, qi: (bh // H, bh % H, qi, 0)),
            pl.BlockSpec((1, 1, S, DH), lambda bh, qi: (bh // H, bh % H, 0, 0)),
            pl.BlockSpec((1, 1, S, DH), lambda bh, qi: (bh // H, bh % H, 0, 0)),
        ],
        out_specs=pl.BlockSpec((1, 1, BQ, DH), lambda bh, qi: (bh // H, bh % H, qi, 0)),
        out_shape=jax.ShapeDtypeStruct((B, H, S, DH), jnp.bfloat16),
    )(q4, k4, v4)


def _oproj_body(o_ref, xn_ref, w_ref, g_ref, hid_ref):
    acc = _dot(o_ref[0, 0], w_ref[0])
    for h in range(1, H):
        acc = acc + _dot(o_ref[0, h], w_ref[h])
    t = acc + xn_ref[...]
    nrm = lax.rsqrt(jnp.mean(t * t, axis=1, keepdims=True) + EPS)
    hid_ref[...] = t * nrm * g_ref[...]


def _oproj(o4, xn, wo_r, rms2):
    nb_per_b = S // RB
    return pl.pallas_call(
        _oproj_body,
        grid=(NBLK,),
        in_specs=[
            pl.BlockSpec((1, H, RB, DH), lambda i: (i // nb_per_b, 0, i % nb_per_b, 0)),
            pl.BlockSpec((RB, D), lambda i: (i, 0)),
            pl.BlockSpec((H, DH, D), lambda i: (0, 0, 0)),
            pl.BlockSpec((1, D), lambda i: (0, 0)),
        ],
        out_specs=pl.BlockSpec((RB, D), lambda i: (i, 0)),
        out_shape=jax.ShapeDtypeStruct((T, D), jnp.float32),
    )(o4, xn, wo_r, rms2)


def _router_body(h_ref, wr_ref, st_ref, dest_ref, gate_ref, stats_ref, carry_ref):
    k = pl.program_id(0)
    rb = pl.program_id(1)

    @pl.when(jnp.logical_and(k == 0, rb == 0))
    def _init():
        carry_ref[...] = jnp.zeros_like(carry_ref)

    h = h_ref[...]
    logits = _dot(h, wr_ref[...])
    lane = lax.broadcasted_iota(jnp.int32, (RB, 128), 1)
    valid = lane < E
    lg = jnp.where(valid, logits, jnp.float32(-1e30))
    m = jnp.max(lg, axis=1, keepdims=True)
    p = jnp.exp(lg - m) * valid.astype(jnp.float32)
    probs = p / jnp.sum(p, axis=1, keepdims=True)

    m0 = jnp.max(probs, axis=1, keepdims=True)
    oh0 = jnp.logical_and(probs == m0, valid)
    e0 = jnp.min(jnp.where(oh0, lane, 128), axis=1, keepdims=True)
    probs2 = jnp.where(lane == e0, jnp.float32(-1.0), probs)
    m1 = jnp.max(probs2, axis=1, keepdims=True)
    oh1 = probs2 == m1
    e1 = jnp.min(jnp.where(oh1, lane, 128), axis=1, keepdims=True)

    is0 = k == 0
    e_sel = jnp.where(is0, e0, e1)
    g = jnp.where(is0, m0, m1) / (m0 + m1)
    oh = (lane == e_sel).astype(jnp.float32)

    ii = lax.broadcasted_iota(jnp.int32, (RB, RB), 0)
    jj = lax.broadcasted_iota(jnp.int32, (RB, RB), 1)
    tril = (jj <= ii).astype(jnp.float32)
    rcnt = lax.dot_general(tril, oh, (((1,), (0,)), ((), ())),
                           precision=lax.Precision.HIGHEST,
                           preferred_element_type=jnp.float32)
    carry = carry_ref[...]
    pos = jnp.sum((rcnt + carry) * oh, axis=1, keepdims=True) - 1.0
    carry_ref[...] = carry + jnp.sum(oh, axis=0, keepdims=True)

    keep = pos < CAP
    pos_i = jnp.minimum(pos, CAP - 1).astype(jnp.int32)
    dest = jnp.where(keep, e_sel * CAP + pos_i, E * CAP)
    eye = (ii == jj).astype(jnp.float32)
    dest_t = lax.dot_general(dest.astype(jnp.float32), eye,
                             (((0,), (0,)), ((), ())),
                             precision=lax.Precision.HIGHEST,
                             preferred_element_type=jnp.float32)
    dest_ref[...] = (dest_t + 0.5).astype(jnp.int32)[None]
    gate_ref[0] = jnp.broadcast_to(g, (RB, 128))

    @pl.when(jnp.logical_and(k == 1, rb == NBLK - 1))
    def _fin():
        stats_ref[...] = st_ref[...] + jnp.minimum(carry_ref[...], float(CAP))


def _router(hidden, wr_pad, stats_pad):
    return pl.pallas_call(
        _router_body,
        grid=(K, NBLK),
        in_specs=[
            pl.BlockSpec((RB, D), lambda k, rb: (rb, 0)),
            pl.BlockSpec((D, 128), lambda k, rb: (0, 0)),
            pl.BlockSpec((1, 128), lambda k, rb: (0, 0)),
        ],
        out_specs=[
            pl.BlockSpec((1, 1, RB), lambda k, rb: (k * NBLK + rb, 0, 0)),
            pl.BlockSpec((1, RB, 128), lambda k, rb: (k, rb, 0)),
            pl.BlockSpec((1, 128), lambda k, rb: (0, 0)),
        ],
        out_shape=[
            jax.ShapeDtypeStruct((K * NBLK, 1, RB), jnp.int32),
            jax.ShapeDtypeStruct((K, T, 128), jnp.float32),
            jax.ShapeDtypeStruct((1, 128), jnp.float32),
        ],
        scratch_shapes=[pltpu.VMEM((1, 128), jnp.float32)],
    )(hidden, wr_pad, stats_pad)


_SC_SCRATCH = [
    pltpu.VMEM((NCHUNK, CHUNK), jnp.int32),
    pltpu.VMEM((CHUNK, D), jnp.float32),
    pltpu.SemaphoreType.DMA,
]


def _sc_mesh():
    return plsc.VectorSubcoreMesh(core_axis_name="c", subcore_axis_name="s")


def _sc_dispatch(hidden, dest3):
    def body(hid, dst, buf, idx_v, rows_v, sem):
        wid = lax.axis_index("s") * 2 + lax.axis_index("c")
        pltpu.sync_copy(dst.at[wid], idx_v)
        tok = lax.rem(wid, 16) * CPW
        for c in range(NCHUNK):
            pltpu.sync_copy(hid.at[pl.ds(tok + c * CHUNK, CHUNK)], rows_v)
            pltpu.async_copy(rows_v, buf.at[idx_v.at[c]], sem).wait()

    return pl.kernel(
        body,
        out_type=jax.ShapeDtypeStruct((NSLOT, D), jnp.float32),
        mesh=_sc_mesh(),
        scratch_types=_SC_SCRATCH,
    )(hidden, dest3)


def _sc_combine(outbuf, dest3):
    def body(obuf, dst, gath, idx_v, rows_v, sem):
        wid = lax.axis_index("s") * 2 + lax.axis_index("c")
        pltpu.sync_copy(dst.at[wid], idx_v)
        for c in range(NCHUNK):
            pltpu.async_copy(obuf.at[idx_v.at[c]], rows_v, sem).wait()
            pltpu.sync_copy(rows_v, gath.at[pl.ds(wid * CPW + c * CHUNK, CHUNK)])

    return pl.kernel(
        body,
        out_type=jax.ShapeDtypeStruct((NCH, D), jnp.float32),
        mesh=_sc_mesh(),
        scratch_types=_SC_SCRATCH,
    )(outbuf, dest3)


def _ffn_body(x_ref, w1_ref, w2_ref, o_ref):
    h = jnp.maximum(_dot(x_ref[...].astype(jnp.bfloat16), w1_ref[0]), 0.0)
    y = _dot(h.astype(jnp.bfloat16), w2_ref[0])
    o_ref[...] = jnp.where(pl.program_id(0) == NFB - 1, 0.0, y)


def _ffn(buf, w1, w2):
    return pl.pallas_call(
        _ffn_body,
        grid=(NFB,),
        in_specs=[
            pl.BlockSpec((BC, D), lambda i: (i, 0)),
            pl.BlockSpec((1, D, DFF), lambda i: (jnp.minimum(i * BC // CAP, E - 1), 0, 0)),
            pl.BlockSpec((1, DFF, D), lambda i: (jnp.minimum(i * BC // CAP, E - 1), 0, 0)),
        ],
        out_specs=pl.BlockSpec((BC, D), lambda i: (i, 0)),
        out_shape=jax.ShapeDtypeStruct((NSLOT, D), jnp.float32),
    )(buf, w1, w2)


def _combine_body(h_ref, r0_ref, r1_ref, g0_ref, g1_ref, o_ref):
    g0 = g0_ref[0][:, 0:1]
    g1 = g1_ref[0][:, 0:1]
    o_ref[...] = (2.0 * h_ref[...] + g0 * r0_ref[...] + g1 * r1_ref[...])


def _final(hidden, gathered, gates_b):
    return pl.pallas_call(
        _combine_body,
        grid=(NBLK,),
        in_specs=[
            pl.BlockSpec((RB, D), lambda i: (i, 0)),
            pl.BlockSpec((RB, D), lambda i: (i, 0)),
            pl.BlockSpec((RB, D), lambda i: (i + NBLK, 0)),
            pl.BlockSpec((1, RB, 128), lambda i: (0, i, 0)),
            pl.BlockSpec((1, RB, 128), lambda i: (1, i, 0)),
        ],
        out_specs=pl.BlockSpec((RB, D), lambda i: (i, 0)),
        out_shape=jax.ShapeDtypeStruct((T, D), jnp.float32),
    )(hidden, gathered, gathered, gates_b, gates_b)


def kernel(tokens, stats, rms1_w, rms2_w, Wq, Wk, Wv, Wo, Wr, W1, W2):
    x2d = tokens.reshape(T, D)
    wqkv = jnp.concatenate([Wq, Wk, Wv], axis=1).astype(jnp.bfloat16)
    xn, qkv = _qkv(x2d, wqkv, rms1_w.reshape(1, D))

    q4 = qkv[:, :D].reshape(B, S, H, DH).transpose(0, 2, 1, 3)
    k4 = qkv[:, D:2 * D].reshape(B, S, H, DH).transpose(0, 2, 1, 3)
    v4 = qkv[:, 2 * D:].reshape(B, S, H, DH).transpose(0, 2, 1, 3)
    o4 = _attention(q4, k4, v4)

    wo_r = Wo.reshape(H, DH, D).astype(jnp.bfloat16)
    hidden = _oproj(o4, xn, wo_r, rms2_w.reshape(1, D))

    wr_pad = jnp.pad(Wr, ((0, 0), (0, 128 - E)))
    stats_pad = jnp.pad(stats, (0, 128 - E)).reshape(1, 128)
    dest_c, gates_b, stats_out = _router(hidden, wr_pad, stats_pad)

    dest3 = dest_c.reshape(NW, NCHUNK, CHUNK)
    buf = _sc_dispatch(hidden, dest3)
    outbuf = _ffn(buf, W1.astype(jnp.bfloat16), W2.astype(jnp.bfloat16))
    gathered = _sc_combine(outbuf, dest3)

    out2d = _final(hidden, gathered, gates_b)
    return out2d.reshape(B, S, D), stats_out[0, :E]
